# Initial kernel scaffold; baseline (speedup 1.0000x reference)
import jax
import jax.numpy as jnp
from jax.experimental import pallas as pl


def kernel(a, emb_weight):
    raise NotImplementedError("write your pallas kernel here")


# speedup vs baseline: 6.2553x; 6.2553x over previous
import functools

import jax
import jax.numpy as jnp
from jax import lax
from jax.experimental import pallas as pl
from jax.experimental.pallas import tpu as pltpu
from jax.experimental.pallas import tpu_sc as plsc

N_ACTIONS = 100000
A_DIM = 16
BATCH = 16384
HIST = 200
B_TOTAL = BATCH * HIST

_NC = 2
_NS = 16
_NW = _NC * _NS

_PER_W = B_TOTAL // _NW
_CHUNK = 2048
_N_CHUNKS = _PER_W // _CHUNK


def _embed_kernel(idx_hbm, table_hbm, out_hbm, idx_v, rows_v, sem):
    wid = lax.axis_index("s") * _NC + lax.axis_index("c")
    base_w = wid * _PER_W

    def body(i, carry):
        base = base_w + i * _CHUNK
        pltpu.sync_copy(idx_hbm.at[pl.ds(base, _CHUNK)], idx_v)
        pltpu.async_copy(table_hbm.at[idx_v], rows_v, sem).wait()
        pltpu.sync_copy(rows_v, out_hbm.at[pl.ds(base, _CHUNK)])
        return carry

    lax.fori_loop(0, _N_CHUNKS, body, 0)


@jax.jit
def _embed(a_flat, emb_weight):
    mesh = plsc.VectorSubcoreMesh(core_axis_name="c", subcore_axis_name="s")
    run = pl.kernel(
        _embed_kernel,
        out_type=jax.ShapeDtypeStruct((B_TOTAL, A_DIM), jnp.float32),
        mesh=mesh,
        scratch_types=[
            pltpu.VMEM((_CHUNK,), jnp.int32),
            pltpu.VMEM((_CHUNK, A_DIM), jnp.float32),
            pltpu.SemaphoreType.DMA,
        ],
        compiler_params=pltpu.CompilerParams(use_tc_tiling_on_sc=False),
    )
    return run(a_flat, emb_weight)


def kernel(a, emb_weight):
    a_flat = a.astype(jnp.int32).reshape(B_TOTAL)
    out = _embed(a_flat, emb_weight)
    return out.reshape(BATCH, HIST, A_DIM)


# speedup vs baseline: 6.4479x; 1.0308x over previous
import functools

import jax
import jax.numpy as jnp
from jax import lax
from jax.experimental import pallas as pl
from jax.experimental.pallas import tpu as pltpu
from jax.experimental.pallas import tpu_sc as plsc

N_ACTIONS = 100000
A_DIM = 16
BATCH = 16384
HIST = 200
B_TOTAL = BATCH * HIST

_NC = 2
_NS = 16
_NW = _NC * _NS

_PER_W = B_TOTAL // _NW
_CHUNK = 2560
_N_CHUNKS = _PER_W // _CHUNK
_NBUF = 2


def _embed_kernel(idx_hbm, table_hbm, out_hbm,
                  idx_v0, idx_v1, rows_v0, rows_v1,
                  sem_i0, sem_i1, sem_g0, sem_g1, sem_o0, sem_o1):
    wid = lax.axis_index("s") * _NC + lax.axis_index("c")
    base_w = wid * _PER_W
    idx_v = (idx_v0, idx_v1)
    rows_v = (rows_v0, rows_v1)
    sem_i = (sem_i0, sem_i1)
    sem_g = (sem_g0, sem_g1)
    sem_o = (sem_o0, sem_o1)
    last = _N_CHUNKS - 1

    def start_idx(c, b):
        cc = jnp.minimum(c, last)
        pltpu.async_copy(idx_hbm.at[pl.ds(base_w + cc * _CHUNK, _CHUNK)],
                         idx_v[b], sem_i[b])

    start_idx(0, 0)
    start_idx(1, 1)

    def outer(g, carry):
        for b in range(_NBUF):
            c = g * _NBUF + b
            pltpu.make_async_copy(idx_hbm.at[pl.ds(0, _CHUNK)],
                                  idx_v[b], sem_i[b]).wait()

            @pl.when(c >= _NBUF)
            def _wait_prev_out():
                pltpu.make_async_copy(rows_v[b],
                                      out_hbm.at[pl.ds(0, _CHUNK)],
                                      sem_o[b]).wait()

            gather = pltpu.async_copy(table_hbm.at[idx_v[b]], rows_v[b],
                                      sem_g[b])
            gather.wait()
            start_idx(c + _NBUF, b)
            pltpu.async_copy(rows_v[b],
                             out_hbm.at[pl.ds(base_w + c * _CHUNK, _CHUNK)],
                             sem_o[b])
        return carry

    lax.fori_loop(0, _N_CHUNKS // _NBUF, outer, 0)

    for b in range(_NBUF):
        pltpu.make_async_copy(idx_hbm.at[pl.ds(0, _CHUNK)],
                              idx_v[b], sem_i[b]).wait()
        pltpu.make_async_copy(rows_v[b], out_hbm.at[pl.ds(0, _CHUNK)],
                              sem_o[b]).wait()


@jax.jit
def _embed(a_flat, emb_weight):
    mesh = plsc.VectorSubcoreMesh(core_axis_name="c", subcore_axis_name="s")
    run = pl.kernel(
        _embed_kernel,
        out_type=jax.ShapeDtypeStruct((B_TOTAL, A_DIM), jnp.float32),
        mesh=mesh,
        scratch_types=[
            pltpu.VMEM((_CHUNK,), jnp.int32),
            pltpu.VMEM((_CHUNK,), jnp.int32),
            pltpu.VMEM((_CHUNK, A_DIM), jnp.float32),
            pltpu.VMEM((_CHUNK, A_DIM), jnp.float32),
            pltpu.SemaphoreType.DMA,
            pltpu.SemaphoreType.DMA,
            pltpu.SemaphoreType.DMA,
            pltpu.SemaphoreType.DMA,
            pltpu.SemaphoreType.DMA,
            pltpu.SemaphoreType.DMA,
        ],
        compiler_params=pltpu.CompilerParams(use_tc_tiling_on_sc=False),
    )
    return run(a_flat, emb_weight)


def kernel(a, emb_weight):
    a_flat = a.astype(jnp.int32).reshape(B_TOTAL)
    out = _embed(a_flat, emb_weight)
    return out.reshape(BATCH, HIST, A_DIM)
